# baseline (speedup 1.0000x reference)
import jax
import jax.numpy as jnp
from jax.experimental import pallas as pl
from jax.experimental.pallas import tpu as pltpu

_TOKEN_DIM = 1024
_CODE_DIM = 64
_HALF = (8 - 1) / 2.0
_BLK = 512


def _fsq_body(x_ref, win_ref, bin_ref, wout_ref, bout_ref, out_ref, loss_ref):
    z = jnp.dot(x_ref[...], win_ref[...], preferred_element_type=jnp.float32)
    z = z + bin_ref[...]
    z_b = jnp.tanh(z) * _HALF
    z_q = jnp.round(z_b)
    d = z_q - z_b

    @pl.when(pl.program_id(0) == 0)
    def _init():
        loss_ref[0, 0] = 0.0

    loss_ref[0, 0] += jnp.sum(d * d)

    out_ref[...] = (
        jnp.dot(z_q * (1.0 / _HALF), wout_ref[...],
                preferred_element_type=jnp.float32)
        + bout_ref[...]
    )


def kernel(inputs_embeds, W_in, b_in, W_out, b_out):
    b, s, dm = inputs_embeds.shape
    n_tok = b * s
    x = inputs_embeds.reshape(n_tok, dm)
    grid = (n_tok // _BLK,)

    out, loss_sum = pl.pallas_call(
        _fsq_body,
        grid=grid,
        in_specs=[
            pl.BlockSpec((_BLK, dm), lambda i: (i, 0)),
            pl.BlockSpec((dm, _CODE_DIM), lambda i: (0, 0)),
            pl.BlockSpec((1, _CODE_DIM), lambda i: (0, 0)),
            pl.BlockSpec((_CODE_DIM, dm), lambda i: (0, 0)),
            pl.BlockSpec((1, dm), lambda i: (0, 0)),
        ],
        out_specs=[
            pl.BlockSpec((_BLK, dm), lambda i: (i, 0)),
            pl.BlockSpec(memory_space=pltpu.SMEM, block_shape=(1, 1),
                         index_map=lambda i: (0, 0)),
        ],
        out_shape=[
            jax.ShapeDtypeStruct((n_tok, dm), jnp.float32),
            jax.ShapeDtypeStruct((1, 1), jnp.float32),
        ],
        compiler_params=pltpu.CompilerParams(
            dimension_semantics=("arbitrary",),
        ),
    )(x, W_in, b_in.reshape(1, _CODE_DIM), W_out, b_out.reshape(1, dm))

    vq_loss = (0.35 / (n_tok * _CODE_DIM)) * loss_sum[0, 0]
    return (out.reshape(b, s, dm), vq_loss)


# speedup vs baseline: 1.1508x; 1.1508x over previous
import jax
import jax.numpy as jnp
from jax.experimental import pallas as pl
from jax.experimental.pallas import tpu as pltpu

_TOKEN_DIM = 1024
_CODE_DIM = 64
_HALF = (8 - 1) / 2.0
_BLK = 1024


def _fsq_body(x_ref, win_ref, bin_ref, wout_ref, bout_ref, out_ref, loss_ref):
    z = jnp.dot(x_ref[...], win_ref[...], preferred_element_type=jnp.float32)
    z = z + bin_ref[...]
    z_b = jnp.tanh(z) * _HALF
    z_q = jnp.round(z_b)
    d = z_q - z_b

    @pl.when(pl.program_id(0) == 0)
    def _init():
        loss_ref[0, 0] = 0.0

    loss_ref[0, 0] += jnp.sum(d * d)

    out_ref[...] = (
        jnp.dot(z_q * (1.0 / _HALF), wout_ref[...],
                preferred_element_type=jnp.float32)
        + bout_ref[...]
    )


def kernel(inputs_embeds, W_in, b_in, W_out, b_out):
    b, s, dm = inputs_embeds.shape
    n_tok = b * s
    x = inputs_embeds.reshape(n_tok, dm)
    grid = (n_tok // _BLK,)

    out, loss_sum = pl.pallas_call(
        _fsq_body,
        grid=grid,
        in_specs=[
            pl.BlockSpec((_BLK, dm), lambda i: (i, 0)),
            pl.BlockSpec((dm, _CODE_DIM), lambda i: (0, 0)),
            pl.BlockSpec((1, _CODE_DIM), lambda i: (0, 0)),
            pl.BlockSpec((_CODE_DIM, dm), lambda i: (0, 0)),
            pl.BlockSpec((1, dm), lambda i: (0, 0)),
        ],
        out_specs=[
            pl.BlockSpec((_BLK, dm), lambda i: (i, 0)),
            pl.BlockSpec(memory_space=pltpu.SMEM, block_shape=(1, 1),
                         index_map=lambda i: (0, 0)),
        ],
        out_shape=[
            jax.ShapeDtypeStruct((n_tok, dm), jnp.float32),
            jax.ShapeDtypeStruct((1, 1), jnp.float32),
        ],
        compiler_params=pltpu.CompilerParams(
            dimension_semantics=("arbitrary",),
        ),
    )(x, W_in, b_in.reshape(1, _CODE_DIM), W_out, b_out.reshape(1, dm))

    vq_loss = (0.35 / (n_tok * _CODE_DIM)) * loss_sum[0, 0]
    return (out.reshape(b, s, dm), vq_loss)


# speedup vs baseline: 1.1858x; 1.0304x over previous
import jax
import jax.numpy as jnp
from jax.experimental import pallas as pl
from jax.experimental.pallas import tpu as pltpu

_TOKEN_DIM = 1024
_CODE_DIM = 64
_HALF = (8 - 1) / 2.0
_BLK = 2048


def _fsq_body(x_ref, win_ref, bin_ref, wout_ref, bout_ref, out_ref, loss_ref):
    z = jnp.dot(x_ref[...], win_ref[...], preferred_element_type=jnp.float32)
    z = z + bin_ref[...]
    z_b = jnp.tanh(z) * _HALF
    z_q = jnp.round(z_b)
    d = z_q - z_b

    @pl.when(pl.program_id(0) == 0)
    def _init():
        loss_ref[0, 0] = 0.0

    loss_ref[0, 0] += jnp.sum(d * d)

    out_ref[...] = (
        jnp.dot(z_q * (1.0 / _HALF), wout_ref[...],
                preferred_element_type=jnp.float32)
        + bout_ref[...]
    )


def kernel(inputs_embeds, W_in, b_in, W_out, b_out):
    b, s, dm = inputs_embeds.shape
    n_tok = b * s
    x = inputs_embeds.reshape(n_tok, dm)
    grid = (n_tok // _BLK,)

    out, loss_sum = pl.pallas_call(
        _fsq_body,
        grid=grid,
        in_specs=[
            pl.BlockSpec((_BLK, dm), lambda i: (i, 0)),
            pl.BlockSpec((dm, _CODE_DIM), lambda i: (0, 0)),
            pl.BlockSpec((1, _CODE_DIM), lambda i: (0, 0)),
            pl.BlockSpec((_CODE_DIM, dm), lambda i: (0, 0)),
            pl.BlockSpec((1, dm), lambda i: (0, 0)),
        ],
        out_specs=[
            pl.BlockSpec((_BLK, dm), lambda i: (i, 0)),
            pl.BlockSpec(memory_space=pltpu.SMEM, block_shape=(1, 1),
                         index_map=lambda i: (0, 0)),
        ],
        out_shape=[
            jax.ShapeDtypeStruct((n_tok, dm), jnp.float32),
            jax.ShapeDtypeStruct((1, 1), jnp.float32),
        ],
        compiler_params=pltpu.CompilerParams(
            dimension_semantics=("arbitrary",),
        ),
    )(x, W_in, b_in.reshape(1, _CODE_DIM), W_out, b_out.reshape(1, dm))

    vq_loss = (0.35 / (n_tok * _CODE_DIM)) * loss_sum[0, 0]
    return (out.reshape(b, s, dm), vq_loss)
